# Initial kernel scaffold; baseline (speedup 1.0000x reference)
import functools

import jax
import jax.numpy as jnp
from jax import lax
from jax.experimental import pallas as pl
from jax.experimental.pallas import tpu as pltpu
from jax.experimental.pallas import tpu_sc as plsc

_VOCAB = 1_000_000
_D = 32
_B = 4096
_L = 200


_VB = 16384


def _scores_body(tab_ref, w_ref, b_ref, out_ref):
    prod = tab_ref[...] * w_ref[...]
    out_ref[...] = jnp.sum(prod, axis=1) + b_ref[0]


def _scores(table, fc_w, fc_b):
    grid = (_VOCAB + _VB - 1) // _VB
    return pl.pallas_call(
        _scores_body,
        grid=(grid,),
        in_specs=[
            pl.BlockSpec((_VB, _D), lambda i: (i, 0)),
            pl.BlockSpec((1, _D), lambda i: (0, 0)),
            pl.BlockSpec(memory_space=pltpu.SMEM),
        ],
        out_specs=pl.BlockSpec((_VB,), lambda i: (i,)),
        out_shape=jax.ShapeDtypeStruct((_VOCAB,), jnp.float32),
    )(table, fc_w, fc_b)


_NC = 2
_NS = 16
_NW = _NC * _NS
_ROWS_PW = _B // _NW
_TOK_PW = _ROWS_PW * _L


def _pool_body(text_hbm, scores_hbm, out_hbm, idx_v, gat_v, out_v, sem):
    wid = lax.axis_index("s") * _NC + lax.axis_index("c")
    tbase = wid * _TOK_PW

    pltpu.sync_copy(text_hbm.at[pl.ds(tbase, _TOK_PW)], idx_v)
    pltpu.async_copy(scores_hbm.at[idx_v], gat_v, sem).wait()

    lane = lax.iota(jnp.int32, 16)
    rowbase = lane * _L

    def grp(g, carry):
        def tok(l, acc):
            idx = rowbase + g * (16 * _L) + l
            return acc + plsc.load_gather(gat_v, [idx])

        acc = lax.fori_loop(0, _L, tok, jnp.zeros((16,), jnp.float32))
        pooled = acc * (1.0 / _L)
        out_v[pl.ds(g * 16, 16)] = 1.0 / (1.0 + jnp.exp(-pooled))
        return carry

    lax.fori_loop(0, _ROWS_PW // 16, grp, 0)
    pltpu.sync_copy(out_v, out_hbm.at[pl.ds(wid * _ROWS_PW, _ROWS_PW)])


def _pool(text_flat, scores):
    mesh = plsc.VectorSubcoreMesh(core_axis_name="c", subcore_axis_name="s")
    kern = functools.partial(
        pl.kernel,
        mesh=mesh,
        out_type=jax.ShapeDtypeStruct((_B,), jnp.float32),
        scratch_types=[
            pltpu.VMEM((_TOK_PW,), jnp.int32),
            pltpu.VMEM((_TOK_PW,), jnp.float32),
            pltpu.VMEM((_ROWS_PW,), jnp.float32),
            pltpu.SemaphoreType.DMA,
        ],
    )(_pool_body)
    return kern(text_flat, scores)


def kernel(text, table, fc_w, fc_b):
    text_flat = text.reshape(-1).astype(jnp.int32)
    scores = _scores(table, fc_w, fc_b)
    probs = _pool(text_flat, scores)
    return probs.reshape(_B, 1)


# speedup vs baseline: 1.8541x; 1.8541x over previous
import functools

import jax
import jax.numpy as jnp
from jax import lax
from jax.experimental import pallas as pl
from jax.experimental.pallas import tpu as pltpu
from jax.experimental.pallas import tpu_sc as plsc

_VOCAB = 1_000_000
_D = 32
_B = 4096
_L = 200


_R = _VOCAB // 4
_RB = 25000


def _scores_body(tab_ref, w4_ref, b_ref, out_ref):
    out_ref[...] = jax.lax.dot_general(
        tab_ref[...], w4_ref[...],
        dimension_numbers=(((1,), (0,)), ((), ())),
        preferred_element_type=jnp.float32,
    ) + b_ref[0]


def _scores(table, fc_w, fc_b):
    t128 = table.reshape(_R, 128)
    w4 = jnp.kron(jnp.eye(4, dtype=jnp.float32), fc_w.reshape(_D, 1))
    s4 = pl.pallas_call(
        _scores_body,
        grid=(_R // _RB,),
        in_specs=[
            pl.BlockSpec((_RB, 128), lambda i: (i, 0)),
            pl.BlockSpec((128, 4), lambda i: (0, 0)),
            pl.BlockSpec(memory_space=pltpu.SMEM),
        ],
        out_specs=pl.BlockSpec((_RB, 4), lambda i: (i, 0)),
        out_shape=jax.ShapeDtypeStruct((_R, 4), jnp.float32),
    )(t128, w4, fc_b)
    return s4.reshape(_VOCAB)


_NC = 2
_NS = 16
_NW = _NC * _NS
_ROWS_PW = _B // _NW
_TOK_PW = _ROWS_PW * _L


def _pool_body(textT_hbm, scores_hbm, out_hbm, idx_v, gat_v, out_v, sem):
    wid = lax.axis_index("s") * _NC + lax.axis_index("c")
    base = wid * _ROWS_PW

    pltpu.sync_copy(textT_hbm.at[:, pl.ds(base, _ROWS_PW)], idx_v)

    def fire(l, c):
        pltpu.async_copy(scores_hbm.at[idx_v.at[l]], gat_v.at[l], sem)
        return c

    lax.fori_loop(0, _L, fire, 0)

    def drain(l, c):
        pltpu.make_async_copy(scores_hbm.at[idx_v.at[0]], gat_v.at[l], sem).wait()
        return c

    lax.fori_loop(0, _L, drain, 0)

    def grp(j, carry):
        def tok(l, acc):
            return acc + gat_v[l, pl.ds(j * 16, 16)]

        acc = lax.fori_loop(0, _L, tok, jnp.zeros((16,), jnp.float32))
        pooled = acc * (1.0 / _L)
        out_v[pl.ds(j * 16, 16)] = 1.0 / (1.0 + jnp.exp(-pooled))
        return carry

    lax.fori_loop(0, _ROWS_PW // 16, grp, 0)
    pltpu.sync_copy(out_v, out_hbm.at[pl.ds(base, _ROWS_PW)])


def _pool(textT, scores):
    mesh = plsc.VectorSubcoreMesh(core_axis_name="c", subcore_axis_name="s")
    kern = functools.partial(
        pl.kernel,
        mesh=mesh,
        out_type=jax.ShapeDtypeStruct((_B,), jnp.float32),
        scratch_types=[
            pltpu.VMEM((_L, _ROWS_PW), jnp.int32),
            pltpu.VMEM((_L, _ROWS_PW), jnp.float32),
            pltpu.VMEM((_ROWS_PW,), jnp.float32),
            pltpu.SemaphoreType.DMA,
        ],
    )(_pool_body)
    return kern(textT, scores)


def kernel(text, table, fc_w, fc_b):
    textT = text.astype(jnp.int32).T
    scores = _scores(table, fc_w, fc_b)
    probs = _pool(textT, scores)
    return probs.reshape(_B, 1)


# speedup vs baseline: 2.2051x; 1.1893x over previous
import functools

import jax
import jax.numpy as jnp
from jax import lax
from jax.experimental import pallas as pl
from jax.experimental.pallas import tpu as pltpu
from jax.experimental.pallas import tpu_sc as plsc

_VOCAB = 1_000_000
_D = 32
_B = 4096
_L = 200

_NC = 2
_NS = 16
_NW = _NC * _NS
_ROWS_PW = _B // _NW
_TOK_PW = _ROWS_PW * _L
_C0 = 128
_C1 = _L - _C0


def _pool_body(textf_hbm, table_hbm, out_hbm, idx_v, gat_v, out_v, sem_a, sem_b):
    wid = lax.axis_index("s") * _NC + lax.axis_index("c")
    base = wid * _ROWS_PW

    pltpu.sync_copy(textf_hbm.at[pl.ds(base * _L, _TOK_PW)], idx_v)
    zero = jnp.zeros((16,), jnp.float32)

    def fire(r, buf, sem):
        off = r * _L
        pltpu.async_copy(table_hbm.at[idx_v.at[pl.ds(off, _C0)]],
                         gat_v.at[buf, pl.ds(0, _C0), :], sem)
        pltpu.async_copy(table_hbm.at[idx_v.at[pl.ds(off + _C0, _C1)]],
                         gat_v.at[buf, pl.ds(_C0, _C1), :], sem)

    def drain(buf, sem):
        pltpu.make_async_copy(table_hbm.at[idx_v.at[pl.ds(0, _C0)]],
                              gat_v.at[buf, pl.ds(0, _C0), :], sem).wait()
        pltpu.make_async_copy(table_hbm.at[idx_v.at[pl.ds(0, _C1)]],
                              gat_v.at[buf, pl.ds(_C0, _C1), :], sem).wait()

    def reduce_row(r, buf):
        def tok(l, accs):
            a0, a1 = accs
            return (a0 + gat_v[buf, l, pl.ds(0, 16)],
                    a1 + gat_v[buf, l, pl.ds(16, 16)])

        a0, a1 = lax.fori_loop(0, _L, tok, (zero, zero))
        out_v[pl.ds(r * _D, 16)] = a0
        out_v[pl.ds(r * _D + 16, 16)] = a1

    fire(0, 0, sem_a)

    def pair(t, carry):
        r0 = 2 * t
        fire(r0 + 1, 1, sem_b)
        drain(0, sem_a)
        reduce_row(r0, 0)

        @pl.when(r0 + 2 < _ROWS_PW)
        def _():
            fire(r0 + 2, 0, sem_a)

        drain(1, sem_b)
        reduce_row(r0 + 1, 1)
        return carry

    lax.fori_loop(0, _ROWS_PW // 2, pair, 0)
    pltpu.sync_copy(out_v, out_hbm.at[pl.ds(base * _D, _ROWS_PW * _D)])


def _pool(text_flat, table):
    mesh = plsc.VectorSubcoreMesh(core_axis_name="c", subcore_axis_name="s")
    kern = functools.partial(
        pl.kernel,
        mesh=mesh,
        compiler_params=pltpu.CompilerParams(use_tc_tiling_on_sc=False),
        out_type=jax.ShapeDtypeStruct((_B * _D,), jnp.float32),
        scratch_types=[
            pltpu.VMEM((_TOK_PW,), jnp.int32),
            pltpu.VMEM((2, _L, _D), jnp.float32),
            pltpu.VMEM((_ROWS_PW * _D,), jnp.float32),
            pltpu.SemaphoreType.DMA,
            pltpu.SemaphoreType.DMA,
        ],
    )(_pool_body)
    return kern(text_flat, table)


_HR = _B * _D // 128


def _head_body(x_ref, w4_ref, b_ref, out_ref):
    y = jax.lax.dot_general(
        x_ref[...], w4_ref[...],
        dimension_numbers=(((1,), (0,)), ((), ())),
        preferred_element_type=jnp.float32,
    ) + b_ref[0]
    out_ref[...] = 1.0 / (1.0 + jnp.exp(-y))


def _head(pooled2d, fc_w, fc_b):
    w4 = jnp.kron(jnp.eye(4, dtype=jnp.float32),
                  fc_w.reshape(_D, 1) * (1.0 / _L))
    return pl.pallas_call(
        _head_body,
        in_specs=[
            pl.BlockSpec((_HR, 128), lambda: (0, 0)),
            pl.BlockSpec((128, 4), lambda: (0, 0)),
            pl.BlockSpec(memory_space=pltpu.SMEM),
        ],
        out_specs=pl.BlockSpec((_HR, 4), lambda: (0, 0)),
        out_shape=jax.ShapeDtypeStruct((_HR, 4), jnp.float32),
    )(pooled2d, w4, fc_b)


def kernel(text, table, fc_w, fc_b):
    text_flat = text.astype(jnp.int32).reshape(-1)
    pooled = _pool(text_flat, table)
    probs4 = _head(pooled.reshape(_HR, 128), fc_w, fc_b)
    return probs4.reshape(_B, 1)
